# baseline (speedup 1.0000x reference)
import jax
import jax.numpy as jnp
from jax import lax
from jax.experimental import pallas as pl
from jax.experimental.pallas import tpu as pltpu
from jax.experimental.pallas import tpu_sc as plsc

NC = 2
NS = 16
NW = NC * NS
BATCH = 1024
SEQ_LEN = 200
EMBED_DIM = 128
ROWS = BATCH * SEQ_LEN
ROWS_PER_W = ROWS // NW
CHUNK = SEQ_LEN
NCHUNK = ROWS_PER_W // CHUNK
NBUF = 3
NP = 10
IDX_MINOR = 100
IDX_ROWS_PER_W = ROWS_PER_W // IDX_MINOR
NGRP = EMBED_DIM // 16


def _sc_body(idx_hbm, tok_hbm, pos_hbm, out_hbm,
             idx_v, pos_v, bufs, gsems, osems):
    wid = lax.axis_index("s") * NC + lax.axis_index("c")
    base_row = wid * ROWS_PER_W
    idx_base = wid * IDX_ROWS_PER_W

    pltpu.sync_copy(pos_hbm, pos_v)
    pltpu.sync_copy(idx_hbm.at[pl.ds(idx_base, IDX_ROWS_PER_W)], idx_v)

    def issue_gather(c, b):
        pltpu.async_copy(tok_hbm.at[idx_v.at[2 * c]],
                         bufs[b].at[pl.ds(0, IDX_MINOR)], gsems[b])
        pltpu.async_copy(tok_hbm.at[idx_v.at[2 * c + 1]],
                         bufs[b].at[pl.ds(IDX_MINOR, IDX_MINOR)], gsems[b])

    def wait_gather(b):
        pltpu.make_async_copy(tok_hbm.at[pl.ds(0, CHUNK)], bufs[b],
                              gsems[b]).wait()

    def wait_out(b):
        pltpu.make_async_copy(bufs[b], out_hbm.at[pl.ds(0, CHUNK)],
                              osems[b]).wait()

    def add_pos(b):
        buf = bufs[b]

        @pl.loop(0, CHUNK, unroll=2)
        def _(r):
            for g in range(NGRP):
                x = pos_v[r, pl.ds(g * 16, 16)]
                plsc.addupdate(buf.at[r, pl.ds(g * 16, 16)], x)

    def consume(c, b):
        wait_gather(b)
        add_pos(b)
        pltpu.async_copy(bufs[b],
                         out_hbm.at[pl.ds(base_row + c * CHUNK, CHUNK)],
                         osems[b])

    issue_gather(0, 0)
    issue_gather(1, 1)

    @pl.loop(0, NP)
    def _(p):
        for b in range(NBUF):
            c = p * NBUF + b
            nb = (b + 2) % NBUF
            if b == 0:
                @pl.when(p > 0)
                def _():
                    wait_out(nb)
            else:
                wait_out(nb)
            issue_gather(c + 2, nb)
            consume(c, b)

    consume(NP * NBUF, 0)
    consume(NP * NBUF + 1, 1)
    for b in range(NBUF):
        wait_out(b)


def kernel(inputs, token_table, pos_table):
    b, l = inputs.shape
    idx = inputs.reshape(-1, IDX_MINOR).astype(jnp.int32)
    mesh = plsc.VectorSubcoreMesh(core_axis_name="c", subcore_axis_name="s")
    out = pl.kernel(
        _sc_body,
        out_type=jax.ShapeDtypeStruct((ROWS, EMBED_DIM), jnp.float32),
        mesh=mesh,
        scratch_types=[
            pltpu.VMEM((IDX_ROWS_PER_W, IDX_MINOR), jnp.int32),
            pltpu.VMEM((SEQ_LEN, EMBED_DIM), jnp.float32),
            tuple(pltpu.VMEM((CHUNK, EMBED_DIM), jnp.float32)
                  for _ in range(NBUF)),
            tuple(pltpu.SemaphoreType.DMA for _ in range(NBUF)),
            tuple(pltpu.SemaphoreType.DMA for _ in range(NBUF)),
        ],
    )(idx, token_table, pos_table)
    return out.reshape(b, l, EMBED_DIM)


# speedup vs baseline: 1.1401x; 1.1401x over previous
import jax
import jax.numpy as jnp
from jax import lax
from jax.experimental import pallas as pl
from jax.experimental.pallas import tpu as pltpu
from jax.experimental.pallas import tpu_sc as plsc

NC = 2
NS = 16
NW = NC * NS
BATCH = 1024
SEQ_LEN = 200
EMBED_DIM = 128
ROWS = BATCH * SEQ_LEN
ROWS_PER_W = ROWS // NW
CHUNK = SEQ_LEN
SUPER = 2 * CHUNK
NSUP = ROWS_PER_W // SUPER
IDX_MINOR = 100
IDX_PER_SUP = SUPER // IDX_MINOR
NGRP = EMBED_DIM // 16


def _sc_body(idx_hbm, tok_hbm, pos_hbm, out_hbm,
             ibufs, pos_v, bufs, isems, gsems, osems):
    wid = lax.axis_index("s") * NC + lax.axis_index("c")
    base_row = wid * ROWS_PER_W
    idx_base = wid * (NSUP * IDX_PER_SUP)

    pltpu.sync_copy(pos_hbm, pos_v)

    def issue_idx(s, q):
        pltpu.async_copy(idx_hbm.at[pl.ds(idx_base + s * IDX_PER_SUP,
                                          IDX_PER_SUP)],
                         ibufs[q], isems[q])

    def wait_idx(q):
        pltpu.make_async_copy(idx_hbm.at[pl.ds(0, IDX_PER_SUP)], ibufs[q],
                              isems[q]).wait()

    def issue_gathers(q):
        for half in range(2):
            dst = bufs[2 * q + half]
            pltpu.async_copy(tok_hbm.at[ibufs[q].at[2 * half]],
                             dst.at[pl.ds(0, IDX_MINOR)], gsems[q])
            pltpu.async_copy(tok_hbm.at[ibufs[q].at[2 * half + 1]],
                             dst.at[pl.ds(IDX_MINOR, IDX_MINOR)], gsems[q])

    def wait_gathers(q):
        for half in range(2):
            pltpu.make_async_copy(tok_hbm.at[pl.ds(0, CHUNK)],
                                  bufs[2 * q + half], gsems[q]).wait()

    def wait_outs(q):
        for half in range(2):
            pltpu.make_async_copy(bufs[2 * q + half],
                                  out_hbm.at[pl.ds(0, CHUNK)],
                                  osems[q]).wait()

    def add_pos_pair(q):
        buf_a = bufs[2 * q]
        buf_b = bufs[2 * q + 1]

        @plsc.parallel_loop(0, CHUNK, unroll=2)
        def _(r):
            for g in range(NGRP):
                x = pos_v[r, pl.ds(g * 16, 16)]
                plsc.addupdate(buf_a.at[r, pl.ds(g * 16, 16)], x)
                plsc.addupdate(buf_b.at[r, pl.ds(g * 16, 16)], x)

    def scatter(s, q):
        for half in range(2):
            pltpu.async_copy(
                bufs[2 * q + half],
                out_hbm.at[pl.ds(base_row + s * SUPER + half * CHUNK, CHUNK)],
                osems[q])

    pltpu.sync_copy(idx_hbm.at[pl.ds(idx_base, IDX_PER_SUP)], ibufs[0])
    issue_gathers(0)
    issue_idx(1, 1)

    @pl.loop(0, NSUP // 2)
    def _(p):
        for q in range(2):
            s = 2 * p + q
            nq = 1 - q
            if q == 0:
                @pl.when(p > 0)
                def _():
                    wait_outs(nq)
                wait_idx(nq)
                issue_gathers(nq)
            else:
                @pl.when(p < NSUP // 2 - 1)
                def _():
                    wait_outs(nq)
                    wait_idx(nq)
                    issue_gathers(nq)

            wait_gathers(q)
            @pl.when(p < NSUP // 2 - 1)
            def _():
                issue_idx(s + 2, q)

            add_pos_pair(q)
            scatter(s, q)

    wait_outs(0)
    wait_outs(1)


def kernel(inputs, token_table, pos_table):
    b, l = inputs.shape
    idx = inputs.reshape(-1, IDX_MINOR).astype(jnp.int32)
    mesh = plsc.VectorSubcoreMesh(core_axis_name="c", subcore_axis_name="s")
    out = pl.kernel(
        _sc_body,
        out_type=jax.ShapeDtypeStruct((ROWS, EMBED_DIM), jnp.float32),
        mesh=mesh,
        scratch_types=[
            tuple(pltpu.VMEM((IDX_PER_SUP, IDX_MINOR), jnp.int32)
                  for _ in range(2)),
            pltpu.VMEM((SEQ_LEN, EMBED_DIM), jnp.float32),
            tuple(pltpu.VMEM((CHUNK, EMBED_DIM), jnp.float32)
                  for _ in range(4)),
            tuple(pltpu.SemaphoreType.DMA for _ in range(2)),
            tuple(pltpu.SemaphoreType.DMA for _ in range(2)),
            tuple(pltpu.SemaphoreType.DMA for _ in range(2)),
        ],
    )(idx, token_table, pos_table)
    return out.reshape(b, l, EMBED_DIM)


# speedup vs baseline: 1.1688x; 1.0252x over previous
import jax
import jax.numpy as jnp
from jax import lax
from jax.experimental import pallas as pl
from jax.experimental.pallas import tpu as pltpu
from jax.experimental.pallas import tpu_sc as plsc

NC = 2
NS = 16
NW = NC * NS
BATCH = 1024
SEQ_LEN = 200
EMBED_DIM = 128
ROWS = BATCH * SEQ_LEN
ROWS_PER_W = ROWS // NW
SUPER = 2 * SEQ_LEN
NSUP = ROWS_PER_W // SUPER
IDX_MINOR = 100
IDX_PER_SUP = SUPER // IDX_MINOR
NGRP = EMBED_DIM // 16


def _sc_body(idx_hbm, tok_hbm, pos_hbm, out_hbm,
             ibufs, pos_v, bufs, psem, isems, gsems, osems):
    wid = lax.axis_index("s") * NC + lax.axis_index("c")
    base_row = wid * ROWS_PER_W
    idx_base = wid * (NSUP * IDX_PER_SUP)

    def issue_idx(s, q):
        pltpu.async_copy(idx_hbm.at[pl.ds(idx_base + s * IDX_PER_SUP,
                                          IDX_PER_SUP)],
                         ibufs[q], isems[q])

    def wait_idx(q):
        pltpu.make_async_copy(idx_hbm.at[pl.ds(0, IDX_PER_SUP)], ibufs[q],
                              isems[q]).wait()

    def issue_gathers(q):
        for j in range(IDX_PER_SUP):
            pltpu.async_copy(tok_hbm.at[ibufs[q].at[j]],
                             bufs[q].at[pl.ds(j * IDX_MINOR, IDX_MINOR)],
                             gsems[q])

    def wait_gathers(q):
        pltpu.make_async_copy(tok_hbm.at[pl.ds(0, SUPER)], bufs[q],
                              gsems[q]).wait()

    def wait_out(q):
        pltpu.make_async_copy(bufs[q], out_hbm.at[pl.ds(0, SUPER)],
                              osems[q]).wait()

    def add_pos(q):
        buf = bufs[q]

        @plsc.parallel_loop(0, SEQ_LEN, unroll=2)
        def _(r):
            for g in range(NGRP):
                x = pos_v[r, pl.ds(g * 16, 16)]
                plsc.addupdate(buf.at[r, pl.ds(g * 16, 16)], x)
                plsc.addupdate(buf.at[SEQ_LEN + r, pl.ds(g * 16, 16)], x)

    pltpu.sync_copy(idx_hbm.at[pl.ds(idx_base, IDX_PER_SUP)], ibufs[0])
    issue_gathers(0)
    issue_idx(1, 1)
    pltpu.async_copy(pos_hbm, pos_v, psem).wait()

    @pl.loop(0, NSUP // 2)
    def _(p):
        for q in range(2):
            s = 2 * p + q
            nq = 1 - q
            if q == 0:
                @pl.when(p > 0)
                def _():
                    wait_out(nq)
                wait_idx(nq)
                issue_gathers(nq)
            else:
                @pl.when(p < NSUP // 2 - 1)
                def _():
                    wait_out(nq)
                    wait_idx(nq)
                    issue_gathers(nq)

            wait_gathers(q)
            @pl.when(p < NSUP // 2 - 1)
            def _():
                issue_idx(s + 2, q)

            add_pos(q)
            pltpu.async_copy(bufs[q],
                             out_hbm.at[pl.ds(base_row + s * SUPER, SUPER)],
                             osems[q])

    wait_out(0)
    wait_out(1)


def kernel(inputs, token_table, pos_table):
    b, l = inputs.shape
    idx = inputs.reshape(-1, IDX_MINOR).astype(jnp.int32)
    mesh = plsc.VectorSubcoreMesh(core_axis_name="c", subcore_axis_name="s")
    out = pl.kernel(
        _sc_body,
        out_type=jax.ShapeDtypeStruct((ROWS, EMBED_DIM), jnp.float32),
        mesh=mesh,
        scratch_types=[
            tuple(pltpu.VMEM((IDX_PER_SUP, IDX_MINOR), jnp.int32)
                  for _ in range(2)),
            pltpu.VMEM((SEQ_LEN, EMBED_DIM), jnp.float32),
            tuple(pltpu.VMEM((SUPER, EMBED_DIM), jnp.float32)
                  for _ in range(2)),
            pltpu.SemaphoreType.DMA,
            tuple(pltpu.SemaphoreType.DMA for _ in range(2)),
            tuple(pltpu.SemaphoreType.DMA for _ in range(2)),
            tuple(pltpu.SemaphoreType.DMA for _ in range(2)),
        ],
    )(idx, token_table, pos_table)
    return out.reshape(b, l, EMBED_DIM)


# speedup vs baseline: 1.1735x; 1.0040x over previous
import jax
import jax.numpy as jnp
from jax import lax
from jax.experimental import pallas as pl
from jax.experimental.pallas import tpu as pltpu
from jax.experimental.pallas import tpu_sc as plsc

NC = 2
NS = 16
NW = NC * NS
BATCH = 1024
SEQ_LEN = 200
EMBED_DIM = 128
ROWS = BATCH * SEQ_LEN
ROWS_PER_W = ROWS // NW
SUPER = 2 * SEQ_LEN
NSUP = ROWS_PER_W // SUPER
IDX_MINOR = 100
IDX_PER_SUP = SUPER // IDX_MINOR
NGRP = EMBED_DIM // 16


def _sc_body(idx_hbm, tok_hbm, pos_hbm, out_hbm,
             ibufs, pos_v, bufs, psem, isems, gsems, gsems2, osems):
    wid = lax.axis_index("s") * NC + lax.axis_index("c")
    base_row = wid * ROWS_PER_W
    idx_base = wid * (NSUP * IDX_PER_SUP)

    def issue_idx(s, q):
        pltpu.async_copy(idx_hbm.at[pl.ds(idx_base + s * IDX_PER_SUP,
                                          IDX_PER_SUP)],
                         ibufs[q], isems[q])

    def wait_idx(q):
        pltpu.make_async_copy(idx_hbm.at[pl.ds(0, IDX_PER_SUP)], ibufs[q],
                              isems[q]).wait()

    def issue_gathers(q):
        for j in (0, 2, 1, 3):
            sem = gsems[q] if j % 2 == 0 else gsems2[q]
            pltpu.async_copy(tok_hbm.at[ibufs[q].at[j]],
                             bufs[q].at[pl.ds(j * IDX_MINOR, IDX_MINOR)],
                             sem)

    def wait_gathers_a(q):
        pltpu.make_async_copy(tok_hbm.at[pl.ds(0, SEQ_LEN)],
                              bufs[q].at[pl.ds(0, SEQ_LEN)], gsems[q]).wait()

    def wait_gathers_b(q):
        pltpu.make_async_copy(tok_hbm.at[pl.ds(0, SEQ_LEN)],
                              bufs[q].at[pl.ds(0, SEQ_LEN)], gsems2[q]).wait()

    def wait_out(q):
        pltpu.make_async_copy(bufs[q], out_hbm.at[pl.ds(0, SUPER)],
                              osems[q]).wait()

    def add_pos_half(q, lo):
        buf = bufs[q]

        @plsc.parallel_loop(lo, lo + IDX_MINOR, unroll=2)
        def _(r):
            for g in range(NGRP):
                x = pos_v[r, pl.ds(g * 16, 16)]
                plsc.addupdate(buf.at[r, pl.ds(g * 16, 16)], x)
                plsc.addupdate(buf.at[SEQ_LEN + r, pl.ds(g * 16, 16)], x)

    pltpu.sync_copy(idx_hbm.at[pl.ds(idx_base, IDX_PER_SUP)], ibufs[0])
    issue_gathers(0)
    issue_idx(1, 1)
    pltpu.async_copy(pos_hbm, pos_v, psem).wait()

    @pl.loop(0, NSUP // 2)
    def _(p):
        for q in range(2):
            s = 2 * p + q
            nq = 1 - q
            if q == 0:
                @pl.when(p > 0)
                def _():
                    wait_out(nq)
                wait_idx(nq)
                issue_gathers(nq)
            else:
                @pl.when(p < NSUP // 2 - 1)
                def _():
                    wait_out(nq)
                    wait_idx(nq)
                    issue_gathers(nq)

            wait_gathers_a(q)
            add_pos_half(q, 0)
            wait_gathers_b(q)
            @pl.when(p < NSUP // 2 - 1)
            def _():
                issue_idx(s + 2, q)

            add_pos_half(q, IDX_MINOR)
            pltpu.async_copy(bufs[q],
                             out_hbm.at[pl.ds(base_row + s * SUPER, SUPER)],
                             osems[q])

    wait_out(0)
    wait_out(1)


def kernel(inputs, token_table, pos_table):
    b, l = inputs.shape
    idx = inputs.reshape(-1, IDX_MINOR).astype(jnp.int32)
    mesh = plsc.VectorSubcoreMesh(core_axis_name="c", subcore_axis_name="s")
    out = pl.kernel(
        _sc_body,
        out_type=jax.ShapeDtypeStruct((ROWS, EMBED_DIM), jnp.float32),
        mesh=mesh,
        scratch_types=[
            tuple(pltpu.VMEM((IDX_PER_SUP, IDX_MINOR), jnp.int32)
                  for _ in range(2)),
            pltpu.VMEM((SEQ_LEN, EMBED_DIM), jnp.float32),
            tuple(pltpu.VMEM((SUPER, EMBED_DIM), jnp.float32)
                  for _ in range(2)),
            pltpu.SemaphoreType.DMA,
            tuple(pltpu.SemaphoreType.DMA for _ in range(2)),
            tuple(pltpu.SemaphoreType.DMA for _ in range(2)),
            tuple(pltpu.SemaphoreType.DMA for _ in range(2)),
            tuple(pltpu.SemaphoreType.DMA for _ in range(2)),
        ],
    )(idx, token_table, pos_table)
    return out.reshape(b, l, EMBED_DIM)
